# Initial kernel scaffold; baseline (speedup 1.0000x reference)
import jax
import jax.numpy as jnp
from jax.experimental import pallas as pl


def kernel(points, inverse_map, voxel_coors, pre_gamma, pre_beta, W0, bn0_gamma, bn0_beta, W1, b1, Wc, bc):
    raise NotImplementedError("write your pallas kernel here")


# speedup vs baseline: 1.4166x; 1.4166x over previous
import functools

import jax
import jax.numpy as jnp
from jax import lax
from jax.experimental import pallas as pl
from jax.experimental.pallas import tpu as pltpu
from jax.experimental.pallas import tpu_sc as plsc

EPS = 1e-5
NC, NS, L = 2, 16, 16
NW = NC * NS
N_PAD = 200704
PPW = N_PAD // NW
KJ = PPW // 128
VOX_PAD = 30720
VPW = VOX_PAD // NW
ZPW = VOX_PAD // NS
F32_MIN = float(jnp.finfo(jnp.float32).min)

_mesh = plsc.VectorSubcoreMesh(
    core_axis_name="c", subcore_axis_name="s", num_cores=NC, num_subcores=NS)


def _wid():
    return lax.axis_index("s") * NC + lax.axis_index("c")


def _iota16():
    return lax.broadcasted_iota(jnp.int32, (L,), 0)


@functools.partial(
    pl.kernel,
    out_type=[
        jax.ShapeDtypeStruct((VOX_PAD, 8), jnp.float32),
        jax.ShapeDtypeStruct((VOX_PAD, 8), jnp.float32),
    ],
    mesh=_mesh,
    compiler_params=pltpu.CompilerParams(needs_layout_passes=False, use_tc_tiling_on_sc=False),
    scratch_types=[
        pltpu.VMEM((PPW, 4), jnp.float32),
        pltpu.VMEM((KJ, 128), jnp.int32),
        pltpu.VMEM((PPW, 8), jnp.float32),
        pltpu.VMEM_SHARED((VOX_PAD, 8), jnp.float32),
    ],
)
def _k1(pts_hbm, inv3d_hbm, zeros_hbm, part0_hbm, part1_hbm,
        pts_v, idx_v, stg_v, acc_sh):
    c = lax.axis_index("c")
    s = lax.axis_index("s")
    wid = _wid()
    base = wid * PPW

    pltpu.sync_copy(pts_hbm.at[pl.ds(base, PPW)], pts_v)
    pltpu.sync_copy(inv3d_hbm.at[wid], idx_v)

    pltpu.sync_copy(zeros_hbm.at[pl.ds(0, ZPW)], acc_sh.at[pl.ds(s * ZPW, ZPW)])
    pltpu.sync_copy(zeros_hbm, stg_v)
    iot = _iota16()
    ones = jnp.ones((L,), jnp.float32)

    def _build(g, _):
        rows = g * L + iot
        for col in range(3):
            v = plsc.load_gather(pts_v, [rows, jnp.full((L,), col, jnp.int32)])
            plsc.store_scatter(stg_v, [rows, jnp.full((L,), col, jnp.int32)], v)
        plsc.store_scatter(stg_v, [rows, jnp.full((L,), 3, jnp.int32)], ones)
        return 0

    lax.fori_loop(0, PPW // L, _build, 0)

    plsc.subcore_barrier()

    def _scat(j, _):
        pltpu.sync_copy(stg_v.at[pl.ds(j * 128, 128)],
                        acc_sh.at[idx_v.at[j]], add=True)
        return 0

    lax.fori_loop(0, KJ, _scat, 0)

    plsc.subcore_barrier()

    @pl.when(c == 0)
    def _():
        pltpu.sync_copy(acc_sh.at[pl.ds(s * ZPW, ZPW)],
                        part0_hbm.at[pl.ds(s * ZPW, ZPW)])

    @pl.when(c == 1)
    def _():
        pltpu.sync_copy(acc_sh.at[pl.ds(s * ZPW, ZPW)],
                        part1_hbm.at[pl.ds(s * ZPW, ZPW)])


@functools.partial(
    pl.kernel,
    out_type=[
        jax.ShapeDtypeStruct((N_PAD, 8), jnp.float32),
        jax.ShapeDtypeStruct((NW * 48, 16), jnp.float32),
    ],
    mesh=_mesh,
    compiler_params=pltpu.CompilerParams(needs_layout_passes=False, use_tc_tiling_on_sc=False),
    scratch_types=[
        pltpu.VMEM((PPW, 4), jnp.float32),
        pltpu.VMEM((KJ, 128), jnp.int32),
        pltpu.VMEM((PPW, 8), jnp.float32),
        pltpu.VMEM((128, 8), jnp.float32),
        pltpu.VMEM((128, 8), jnp.float32),
        pltpu.VMEM((48, 16), jnp.float32),
        pltpu.SemaphoreType.DMA,
        pltpu.SemaphoreType.DMA,
    ],
)
def _k2(pts_hbm, inv3d_hbm, part0_hbm, part1_hbm, f_hbm, mom_hbm,
        pts_v, idx_v, stg_v, grow0_v, grow1_v, macc_v, sem0, sem1):
    wid = _wid()
    base = wid * PPW

    pltpu.sync_copy(pts_hbm.at[pl.ds(base, PPW)], pts_v)
    pltpu.sync_copy(inv3d_hbm.at[wid], idx_v)

    for i in range(48):
        macc_v[i] = jnp.zeros((L,), jnp.float32)

    iot = _iota16()
    half, thalf = jnp.float32(0.5), jnp.float32(1.5)

    def _block(j, _):
        cp0 = pltpu.async_copy(part0_hbm.at[idx_v.at[j]], grow0_v, sem0)
        cp1 = pltpu.async_copy(part1_hbm.at[idx_v.at[j]], grow1_v, sem1)
        cp0.wait()
        cp1.wait()
        for g in range(8):
            rows = g * L + iot
            prow = j * 128 + rows

            def _pcol(col):
                return plsc.load_gather(
                    pts_v, [prow, jnp.full((L,), col, jnp.int32)])

            def _gcol(col):
                a = plsc.load_gather(
                    grow0_v, [rows, jnp.full((L,), col, jnp.int32)])
                b = plsc.load_gather(
                    grow1_v, [rows, jnp.full((L,), col, jnp.int32)])
                return a + b

            x, y, z, w = _pcol(0), _pcol(1), _pcol(2), _pcol(3)
            sx, sy, sz, cnt = _gcol(0), _gcol(1), _gcol(2), _gcol(3)
            inv = jnp.float32(1.0) / cnt
            cx = x - sx * inv
            cy = y - sy * inv
            cz = z - sz * inv
            s2 = x * x + y * y + z * z
            bits = plsc.bitcast(s2, jnp.int32)
            yr = plsc.bitcast(jnp.int32(0x5F3759DF) -
                              lax.shift_right_logical(bits, 1), jnp.float32)
            h = half * s2
            for _ in range(3):
                yr = yr * (thalf - h * yr * yr)
            dist = s2 * yr

            vals = (x, y, z, w, dist, cx, cy, cz)
            for col in range(8):
                plsc.store_scatter(
                    stg_v, [prow, jnp.full((L,), col, jnp.int32)], vals[col])
            k = 8
            for a in range(8):
                plsc.addupdate(macc_v.at[a], vals[a])
                for b in range(a, 8):
                    plsc.addupdate(macc_v.at[k], vals[a] * vals[b])
                    k += 1
        return 0

    lax.fori_loop(0, KJ, _block, 0)

    pltpu.sync_copy(stg_v, f_hbm.at[pl.ds(base, PPW)])
    pltpu.sync_copy(macc_v, mom_hbm.at[pl.ds(wid * 48, 48)])


def _k3_body(f_ref, w0_ref, b0_ref, w1_ref, b1_ref, pf0_ref, pf1_ref):
    fb = f_ref[...]
    pf0 = jnp.maximum(
        lax.dot(fb, w0_ref[...], preferred_element_type=jnp.float32)
        + b0_ref[...], 0.0)
    pf0_ref[...] = pf0
    pf1_ref[...] = (lax.dot(pf0, w1_ref[...],
                            preferred_element_type=jnp.float32) + b1_ref[...])


def _k3(f, w0e, b0e, w1, b1):
    blk = 2048
    grid = N_PAD // blk
    return pl.pallas_call(
        _k3_body,
        grid=(grid,),
        in_specs=[
            pl.BlockSpec((blk, 8), lambda i: (i, 0)),
            pl.BlockSpec((8, 64), lambda i: (0, 0)),
            pl.BlockSpec((1, 64), lambda i: (0, 0)),
            pl.BlockSpec((64, 64), lambda i: (0, 0)),
            pl.BlockSpec((1, 64), lambda i: (0, 0)),
        ],
        out_specs=[
            pl.BlockSpec((blk, 64), lambda i: (i, 0)),
            pl.BlockSpec((blk, 64), lambda i: (i, 0)),
        ],
        out_shape=[
            jax.ShapeDtypeStruct((N_PAD, 64), jnp.float32),
            jax.ShapeDtypeStruct((N_PAD, 64), jnp.float32),
        ],
    )(f, w0e, b0e, w1, b1)


@functools.partial(
    pl.kernel,
    out_type=jax.ShapeDtypeStruct((VOX_PAD, 64), jnp.float32),
    mesh=_mesh,
    compiler_params=pltpu.CompilerParams(needs_layout_passes=False, use_tc_tiling_on_sc=False),
    scratch_types=[
        pltpu.VMEM((PPW,), jnp.int32),
        pltpu.VMEM((PPW + 16,), jnp.int32),
        pltpu.VMEM((PPW + 16,), jnp.int32),
        pltpu.VMEM((VPW, 64), jnp.float32),
        pltpu.VMEM((128, 64), jnp.float32),
        pltpu.SemaphoreType.DMA,
    ],
)
def _k4(inv_hbm, pf1_hbm, minfill_hbm, vmax_hbm,
        scan_v, pid_v, slot_v, acc_v, gbuf_v, sem):
    wid = _wid()
    lo = wid * VPW
    hi = lo + VPW

    pltpu.sync_copy(minfill_hbm, acc_v)

    def _zpid(k, _):
        pid_v[pl.ds(k * L, L)] = jnp.zeros((L,), jnp.int32)
        return 0
    lax.fori_loop(0, (PPW + 16) // L, _zpid, 0)

    iot = _iota16()

    def _chunk(ci, _):
        pltpu.sync_copy(inv_hbm.at[pl.ds(ci * PPW, PPW)], scan_v)

        def _scan(k, cc):
            v = scan_v[pl.ds(k * L, L)]
            m = (v >= lo) & (v < hi)
            mi = m.astype(jnp.int32)
            cum = plsc.cumsum(mi)
            tot = jnp.sum(mi)
            addr = cc + cum - 1
            pid = ci * PPW + k * L + iot
            plsc.store_scatter(pid_v, [addr], pid, mask=m)
            plsc.store_scatter(slot_v, [addr], v - lo, mask=m)
            return cc + tot

        cc = lax.fori_loop(0, PPW // L, _scan, jnp.int32(0))

        nb = (cc + 127) // 128

        def _drain(b, _):
            off = pl.multiple_of(b * 128, 128)
            pltpu.async_copy(
                pf1_hbm.at[pid_v.at[pl.ds(off, 128)]], gbuf_v, sem).wait()
            cb = jnp.minimum(cc - b * 128, 128)

            def _acc1(i, _):
                slot = slot_v[pl.ds(off + i, L)][0]
                for cg in range(4):
                    cur = acc_v[slot, pl.ds(cg * L, L)]
                    row = gbuf_v[i, pl.ds(cg * L, L)]
                    acc_v[slot, pl.ds(cg * L, L)] = jnp.maximum(cur, row)
                return 0

            lax.fori_loop(0, cb, _acc1, 0)
            return 0

        lax.fori_loop(0, nb, _drain, 0)
        return 0

    lax.fori_loop(0, NW, _chunk, 0)

    pltpu.sync_copy(acc_v, vmax_hbm.at[pl.ds(lo, VPW)])


def _k5_body(v_ref, wc_ref, bc_ref, o_ref):
    o_ref[...] = jnp.maximum(
        lax.dot(v_ref[...], wc_ref[...], preferred_element_type=jnp.float32)
        + bc_ref[...], 0.0)


def _k5(vmax, wc, bc):
    blk = 2048
    grid = VOX_PAD // blk
    return pl.pallas_call(
        _k5_body,
        grid=(grid,),
        in_specs=[
            pl.BlockSpec((blk, 64), lambda i: (i, 0)),
            pl.BlockSpec((64, 16), lambda i: (0, 0)),
            pl.BlockSpec((1, 16), lambda i: (0, 0)),
        ],
        out_specs=pl.BlockSpec((blk, 16), lambda i: (i, 0)),
        out_shape=jax.ShapeDtypeStruct((VOX_PAD, 16), jnp.float32),
    )(vmax, wc, bc)


def kernel(points, inverse_map, voxel_coors, pre_gamma, pre_beta, W0,
           bn0_gamma, bn0_beta, W1, b1, Wc, bc):
    n, _ = points.shape
    m = voxel_coors.shape[0]

    pts_pad = jnp.zeros((N_PAD, 4), jnp.float32).at[:n].set(points)
    inv_pad = jnp.full((N_PAD,), m, jnp.int32).at[:n].set(inverse_map)
    inv3d = inv_pad.reshape(NW, KJ, 128)

    zeros8 = jnp.zeros((PPW, 8), jnp.float32)
    part0, part1 = _k1(pts_pad, inv3d, zeros8)
    f, mom = _k2(pts_pad, inv3d, part0, part1)

    momr = jnp.sum(mom.reshape(NW, 48, L), axis=(0, 2))
    sum_f = momr[:8]
    mu = sum_f / n
    pairs = momr[8:44] / n
    idx_a, idx_b = [], []
    for a in range(8):
        for b in range(a, 8):
            idx_a.append(a)
            idx_b.append(b)
    ex2 = jnp.zeros((8, 8), jnp.float32)
    ex2 = ex2.at[jnp.array(idx_a), jnp.array(idx_b)].set(pairs)
    ex2 = ex2 + ex2.T - jnp.diag(jnp.diag(ex2))
    cov = ex2 - jnp.outer(mu, mu)
    var_f = jnp.diag(cov)
    g = pre_gamma / jnp.sqrt(var_f + EPS)
    bfn = pre_beta - mu * g
    w0g = g[:, None] * W0
    mean_y = pre_beta @ W0
    var_y = jnp.einsum('ak,ab,bk->k', w0g, cov, w0g)
    s = bn0_gamma / jnp.sqrt(var_y + EPS)
    t = bn0_beta - mean_y * s
    w0e = w0g * s[None, :]
    b0e = ((bfn @ W0) * s + t).reshape(1, 64)

    pf0p, pf1p = _k3(f, w0e, b0e, W1, b1.reshape(1, 64))
    minfill = jnp.full((VPW, 64), F32_MIN, jnp.float32)
    vmax = _k4(inv_pad, pf1p, minfill)
    vf = _k5(vmax, Wc, bc.reshape(1, 16))

    return (vf[:m], pf0p[:n], pf1p[:n])
